# baseline (speedup 1.0000x reference)
import functools

import jax
import jax.numpy as jnp
from jax import lax
from jax.experimental import pallas as pl
from jax.experimental.pallas import tpu as pltpu
from jax.experimental.pallas import tpu_sc as plsc

B = 16384
T = 4
D = 256
V = 304
F_TC = 8192
B_SC = (B - F_TC) // 2
NC, NS = 2, 16
NW = NC * NS
BPW = B_SC // NW
C = 32
NCHUNK = BPW // C


def _sc_body(idx_hbm, table_hbm, out_hbm,
             idx_v, table_v, out0, out1, sem_o0, sem_o1):
    wid = lax.axis_index("s") * NC + lax.axis_index("c")
    base = wid * BPW
    pltpu.sync_copy(table_hbm, table_v)
    pltpu.sync_copy(idx_hbm.at[pl.ds(base * T, BPW * T)], idx_v)

    outs = (out0, out1)
    sem_o = (sem_o0, sem_o1)

    def odst(c):
        return out_hbm.at[pl.ds(base + c * C, C)]

    iota = lax.iota(jnp.int32, 16)

    def pair_body(p, carry):
        for bu in range(2):
            c = 2 * p + bu
            ob = outs[bu]

            @pl.when(p >= 1)
            def _drain(ob=ob, c=c, bu=bu):
                pltpu.make_async_copy(ob, odst(c - 2), sem_o[bu]).wait()

            @plsc.parallel_loop(0, C, unroll=2)
            def row_body(r, c=c, ob=ob):
                off = (c * C + r) * T
                offv = jnp.full((16,), off, dtype=jnp.int32)
                sidx = []
                for t in range(T):
                    tok = plsc.load_gather(idx_v, [offv + t])
                    sidx.append((tok << 8) + iota)
                gl = V * D - 16 * (D // 16 - 1)
                for d in range(D // 16):
                    dof = 16 * d
                    tv = table_v.at[pl.ds(dof, gl)]
                    a = plsc.load_gather(tv, [sidx[0]])
                    b2 = plsc.load_gather(tv, [sidx[1]])
                    c2 = plsc.load_gather(tv, [sidx[2]])
                    d2 = plsc.load_gather(tv, [sidx[3]])
                    ob[r, pl.ds(dof, 16)] = (a + b2) + (c2 + d2)

            pltpu.async_copy(ob, odst(c), sem_o[bu])
        return carry

    lax.fori_loop(0, NCHUNK // 2, pair_body, 0, unroll=False)

    for c in (NCHUNK - 2, NCHUNK - 1):
        pltpu.make_async_copy(outs[c % 2], odst(c), sem_o[c % 2]).wait()


_sc_embed = functools.partial(
    pl.kernel,
    out_type=jax.ShapeDtypeStruct((B_SC, D), jnp.float32),
    mesh=plsc.VectorSubcoreMesh(core_axis_name="c", subcore_axis_name="s"),
    compiler_params=pltpu.CompilerParams(needs_layout_passes=False),
    scratch_types=[
        pltpu.VMEM((BPW * T,), jnp.int32),
        pltpu.VMEM((V * D,), jnp.float32),
        pltpu.VMEM((C, D), jnp.float32),
        pltpu.VMEM((C, D), jnp.float32),
        pltpu.SemaphoreType.DMA,
        pltpu.SemaphoreType.DMA,
    ],
)(_sc_body)


BLK = 512


def _tc_body(idx_ref, table_ref, out_ref):
    ids = idx_ref[...]
    iota_v = lax.broadcasted_iota(jnp.int32, (BLK, V), 1)
    counts = (ids[:, 0:1] == iota_v).astype(jnp.float32)
    for t in range(1, T):
        counts += (ids[:, t:t + 1] == iota_v).astype(jnp.float32)
    out_ref[...] = jnp.dot(
        counts, table_ref[...], preferred_element_type=jnp.float32
    )


_tc_embed = pl.pallas_call(
    _tc_body,
    grid=(F_TC // BLK,),
    in_specs=[
        pl.BlockSpec((BLK, T), lambda i: (i, 0)),
        pl.BlockSpec((V, D), lambda i: (0, 0)),
    ],
    out_specs=pl.BlockSpec((BLK, D), lambda i: (i, 0)),
    out_shape=jax.ShapeDtypeStruct((F_TC, D), jnp.float32),
)


def kernel(tokens_batch_indices, embedding_weight):
    idx = tokens_batch_indices.astype(jnp.int32)
    tw = embedding_weight.reshape(-1)
    sc_out0 = _sc_embed(idx[F_TC:F_TC + B_SC].reshape(-1), tw)
    sc_out1 = _sc_embed(idx[F_TC + B_SC:].reshape(-1), tw)
    tc_out = _tc_embed(idx[:F_TC], embedding_weight)
    return jnp.concatenate([tc_out, sc_out0, sc_out1], axis=0)


# speedup vs baseline: 1.2441x; 1.2441x over previous
import functools

import jax
import jax.numpy as jnp
from jax import lax
from jax.experimental import pallas as pl
from jax.experimental.pallas import tpu as pltpu
from jax.experimental.pallas import tpu_sc as plsc

B = 16384
T = 4
D = 256
V = 304
F_TC = 8192
B_SC = B - F_TC
NC, NS = 2, 16
NW = NC * NS
BPW = B_SC // NW
C = 64
NCHUNK = BPW // C


def _sc_body(idx_hbm, table_hbm, out_hbm,
             idx_v, table_v, out0, out1, sem_o0, sem_o1):
    wid = lax.axis_index("s") * NC + lax.axis_index("c")
    base = wid * BPW
    pltpu.sync_copy(table_hbm, table_v)
    pltpu.sync_copy(idx_hbm.at[pl.ds(base * T, BPW * T)], idx_v)

    outs = (out0, out1)
    sem_o = (sem_o0, sem_o1)

    def odst(c):
        return out_hbm.at[pl.ds(base + c * C, C)]

    iota = lax.iota(jnp.int32, 16)

    def pair_body(p, carry):
        for bu in range(2):
            c = 2 * p + bu
            ob = outs[bu]

            @pl.when(p >= 1)
            def _drain(ob=ob, c=c, bu=bu):
                pltpu.make_async_copy(ob, odst(c - 2), sem_o[bu]).wait()

            @plsc.parallel_loop(0, C, unroll=2)
            def row_body(r, c=c, ob=ob):
                off = (c * C + r) * T
                offv = jnp.full((16,), off, dtype=jnp.int32)
                sidx = []
                for t in range(T):
                    tok = plsc.load_gather(idx_v, [offv + t])
                    sidx.append((tok << 8) + iota)
                gl = V * D - 16 * (D // 16 - 1)
                for d in range(D // 16):
                    dof = 16 * d
                    tv = table_v.at[pl.ds(dof, gl)]
                    a = plsc.load_gather(tv, [sidx[0]])
                    b2 = plsc.load_gather(tv, [sidx[1]])
                    c2 = plsc.load_gather(tv, [sidx[2]])
                    d2 = plsc.load_gather(tv, [sidx[3]])
                    ob[r, pl.ds(dof, 16)] = (a + b2) + (c2 + d2)

            pltpu.async_copy(ob, odst(c), sem_o[bu])
        return carry

    lax.fori_loop(0, NCHUNK // 2, pair_body, 0, unroll=False)

    for c in (NCHUNK - 2, NCHUNK - 1):
        pltpu.make_async_copy(outs[c % 2], odst(c), sem_o[c % 2]).wait()


_sc_embed = functools.partial(
    pl.kernel,
    out_type=jax.ShapeDtypeStruct((B_SC, D), jnp.float32),
    mesh=plsc.VectorSubcoreMesh(core_axis_name="c", subcore_axis_name="s"),
    compiler_params=pltpu.CompilerParams(needs_layout_passes=False),
    scratch_types=[
        pltpu.VMEM((BPW * T,), jnp.int32),
        pltpu.VMEM((V * D,), jnp.float32),
        pltpu.VMEM((C, D), jnp.float32),
        pltpu.VMEM((C, D), jnp.float32),
        pltpu.SemaphoreType.DMA,
        pltpu.SemaphoreType.DMA,
    ],
)(_sc_body)


BLK = 512


def _tc_body(idx_ref, table_ref, out_ref):
    ids = idx_ref[...]
    iota_v = lax.broadcasted_iota(jnp.int32, (BLK, V), 1)
    counts = (ids[:, 0:1] == iota_v).astype(jnp.float32)
    for t in range(1, T):
        counts += (ids[:, t:t + 1] == iota_v).astype(jnp.float32)
    out_ref[...] = jnp.dot(
        counts, table_ref[...], preferred_element_type=jnp.float32
    )


_tc_embed = pl.pallas_call(
    _tc_body,
    grid=(F_TC // BLK,),
    in_specs=[
        pl.BlockSpec((BLK, T), lambda i: (i, 0)),
        pl.BlockSpec((V, D), lambda i: (0, 0)),
    ],
    out_specs=pl.BlockSpec((BLK, D), lambda i: (i, 0)),
    out_shape=jax.ShapeDtypeStruct((F_TC, D), jnp.float32),
)


def kernel(tokens_batch_indices, embedding_weight):
    idx = tokens_batch_indices.astype(jnp.int32)
    sc_out = _sc_embed(idx[F_TC:].reshape(-1), embedding_weight.reshape(-1))
    tc_out = _tc_embed(idx[:F_TC], embedding_weight)
    return jnp.concatenate([tc_out, sc_out], axis=0)
